# Initial kernel scaffold; baseline (speedup 1.0000x reference)
import jax
import jax.numpy as jnp
from jax.experimental import pallas as pl


def kernel(input):
    raise NotImplementedError("write your pallas kernel here")


# speedup vs baseline: 88.2806x; 88.2806x over previous
import functools

import jax
import jax.numpy as jnp
import numpy as np
from jax import lax
from jax.experimental import pallas as pl
from jax.experimental.pallas import tpu as pltpu
from jax.experimental.pallas import tpu_sc as plsc

_H = _W = 48
_HW = _H * _W
_BC = 12
_L = 16
_NG = _HW // _L
_NC, _NS = 2, 16
_NW = _NC * _NS
_M0 = np.float32(0.05)


def _build_tables():
    h = np.arange(48, 0, -1, dtype=np.int64)
    w = np.arange(1, 49, dtype=np.int64)
    gx = np.tile(w, 48)
    gy = np.repeat(h, 48)
    dx = gx[:, None] - gx[None, :]
    dy = gy[:, None] - gy[None, :]
    d2i = dx * dx + dy * dy
    knn = np.argsort(d2i, axis=-1, kind="stable").astype(np.int32)
    d2f = np.square(np.sqrt(d2i.astype(np.float32))).astype(np.float32)
    knn_d2 = np.take_along_axis(d2f, knn, axis=-1)
    idx_t = np.ascontiguousarray(knn.reshape(_NG, _L, _HW).transpose(0, 2, 1))
    d2_t = np.ascontiguousarray(knn_d2.reshape(_NG, _L, _HW).transpose(0, 2, 1))
    return idx_t, d2_t


_IDX_T, _D2_T = _build_tables()


def _sqrt16(x):
    i = plsc.bitcast(x, jnp.int32)
    i = jnp.int32(0x5F3759DF) - lax.shift_right_logical(i, 1)
    y = plsc.bitcast(i, jnp.float32)
    for _ in range(3):
        y = y * (jnp.float32(1.5) - jnp.float32(0.5) * x * y * y)
    return x * y


def _bounds_tc_body(w_ref, out_ref):
    total = jnp.sum(w_ref[...], axis=1, keepdims=True)
    out_ref[...] = jnp.broadcast_to(total * _M0, (_BC, _L))


def _bounds_tc(weight):
    return pl.pallas_call(
        _bounds_tc_body,
        out_shape=jax.ShapeDtypeStruct((_BC, _L), jnp.float32),
    )(weight)


def _dtm_body(idx_hbm, d2_hbm, w_hbm, bnd_hbm, out_hbm, w_v, idx_v, d2_v,
              bnd_v, out_v):
    wid = lax.axis_index("s") * _NC + lax.axis_index("c")
    pltpu.sync_copy(w_hbm, w_v)
    pltpu.sync_copy(bnd_hbm, bnd_v)

    def _scan_bc(bc, _):
        bcv = jnp.full((_L,), bc, jnp.int32)
        bndv = bnd_v[bc]

        def cond(s):
            j, cumw, cumd, val, fr = s
            return jnp.logical_and(j < _HW, jnp.logical_not(jnp.all(fr)))

        def step(s):
            j, cumw, cumd, val, fr = s
            idxv = idx_v[j]
            d2v = d2_v[j]
            wv = plsc.load_gather(w_v, [bcv, idxv])
            cumw = cumw + wv
            cumd = cumd + d2v * wv
            cand = cumd + d2v * (bndv - cumw)
            val = jnp.where(fr, val, cand)
            fr = jnp.logical_or(fr, cumw >= bndv)
            return j + 1, cumw, cumd, val, fr

        z = jnp.zeros((_L,), jnp.float32)
        _, _, _, val, _ = lax.while_loop(
            cond, step,
            (jnp.int32(0), z, z, z, jnp.zeros((_L,), jnp.bool_)),
        )
        out_v[bc] = _sqrt16(val / bndv)
        return 0

    for t in range(-(-_NG // _NW)):
        g = jnp.minimum(wid + _NW * t, _NG - 1)
        pltpu.sync_copy(idx_hbm.at[g], idx_v)
        pltpu.sync_copy(d2_hbm.at[g], d2_v)
        lax.fori_loop(0, _BC, _scan_bc, 0)
        pltpu.sync_copy(out_v, out_hbm.at[g])


@functools.cache
def _dtm_sc():
    return functools.partial(
        pl.kernel,
        out_type=jax.ShapeDtypeStruct((_NG, _BC, _L), jnp.float32),
        compiler_params=pltpu.CompilerParams(
            needs_layout_passes=False, use_tc_tiling_on_sc=False,
        ),
        mesh=plsc.VectorSubcoreMesh(
            core_axis_name="c", subcore_axis_name="s",
            num_cores=_NC, num_subcores=_NS,
        ),
        scratch_types=[
            pltpu.VMEM((_BC, _HW), jnp.float32),
            pltpu.VMEM((_HW, _L), jnp.int32),
            pltpu.VMEM((_HW, _L), jnp.float32),
            pltpu.VMEM((_BC, _L), jnp.float32),
            pltpu.VMEM((_BC, _L), jnp.float32),
        ],
    )(_dtm_body)


def kernel(input):
    b, c, h, w = input.shape
    weight = input.reshape(_BC, _HW)
    bounds = _bounds_tc(weight)
    dtm = _dtm_sc()(jnp.asarray(_IDX_T), jnp.asarray(_D2_T), weight, bounds)
    return dtm.transpose(1, 0, 2).reshape(b, c, h, w)


# speedup vs baseline: 146.0227x; 1.6541x over previous
import functools

import jax
import jax.numpy as jnp
import numpy as np
from jax import lax
from jax.experimental import pallas as pl
from jax.experimental.pallas import tpu as pltpu
from jax.experimental.pallas import tpu_sc as plsc

_H = _W = 48
_HW = _H * _W
_BC = 12
_L = 16
_NG = _HW // _L
_NC, _NS = 2, 16
_NW = _NC * _NS
_NB = 6
_M0 = np.float32(0.05)


def _build_tables():
    h = np.arange(48, 0, -1, dtype=np.int64)
    w = np.arange(1, 49, dtype=np.int64)
    gx = np.tile(w, 48)
    gy = np.repeat(h, 48)
    dx = gx[:, None] - gx[None, :]
    dy = gy[:, None] - gy[None, :]
    d2i = dx * dx + dy * dy
    knn = np.argsort(d2i, axis=-1, kind="stable").astype(np.int32)
    d2f = np.square(np.sqrt(d2i.astype(np.float32))).astype(np.float32)
    knn_d2 = np.take_along_axis(d2f, knn, axis=-1)
    idx_t = np.ascontiguousarray(knn.reshape(_NG, _L, _HW).transpose(0, 2, 1))
    d2_t = np.ascontiguousarray(knn_d2.reshape(_NG, _L, _HW).transpose(0, 2, 1))
    return idx_t, d2_t


_IDX_T, _D2_T = _build_tables()


def _sqrt16(x):
    i = plsc.bitcast(x, jnp.int32)
    i = jnp.int32(0x5F3759DF) - lax.shift_right_logical(i, 1)
    y = plsc.bitcast(i, jnp.float32)
    for _ in range(3):
        y = y * (jnp.float32(1.5) - jnp.float32(0.5) * x * y * y)
    return x * y


def _bounds_tc_body(w_ref, out_ref):
    total = jnp.sum(w_ref[...], axis=1, keepdims=True)
    out_ref[...] = jnp.broadcast_to(total * _M0, (_BC, _L))


def _bounds_tc(weight):
    return pl.pallas_call(
        _bounds_tc_body,
        out_shape=jax.ShapeDtypeStruct((_BC, _L), jnp.float32),
    )(weight)


def _dtm_body(idx_hbm, d2_hbm, w_hbm, bnd_hbm, out_hbm, w_v, idx_v, d2_v,
              bnd_v, out_v):
    wid = lax.axis_index("s") * _NC + lax.axis_index("c")
    pltpu.sync_copy(w_hbm, w_v)
    pltpu.sync_copy(bnd_hbm, bnd_v)

    def _scan_pass(bcs):
        bcvs = [jnp.full((_L,), bc, jnp.int32) for bc in bcs]
        bndvs = [bnd_v[bc] for bc in bcs]
        nb = len(bcs)

        def cond(s):
            frs = s[1 + 3 * nb:]
            fall = frs[0]
            for f in frs[1:]:
                fall = fall & f
            return jnp.logical_and(s[0] < _HW, jnp.logical_not(jnp.all(fall)))

        def step(s):
            j = s[0]
            cumws = list(s[1:1 + nb])
            cumds = list(s[1 + nb:1 + 2 * nb])
            vals = list(s[1 + 2 * nb:1 + 3 * nb])
            frs = list(s[1 + 3 * nb:])
            idxv = idx_v[j]
            d2v = d2_v[j]
            for i in range(nb):
                wv = plsc.load_gather(w_v, [bcvs[i], idxv])
                cumws[i] = cumws[i] + wv
                cumds[i] = cumds[i] + d2v * wv
                cand = cumds[i] + d2v * (bndvs[i] - cumws[i])
                vals[i] = jnp.where(frs[i], vals[i], cand)
                frs[i] = jnp.logical_or(frs[i], cumws[i] >= bndvs[i])
            return (j + 1, *cumws, *cumds, *vals, *frs)

        z = jnp.zeros((_L,), jnp.float32)
        f0 = jnp.zeros((_L,), jnp.bool_)
        init = (jnp.int32(0), *([z] * (3 * nb)), *([f0] * nb))
        res = lax.while_loop(cond, step, init)
        vals = res[1 + 2 * nb:1 + 3 * nb]
        for i, bc in enumerate(bcs):
            out_v[bc] = _sqrt16(vals[i] / bndvs[i])

    for t in range(-(-_NG // _NW)):
        g = jnp.minimum(wid + _NW * t, _NG - 1)
        pltpu.sync_copy(idx_hbm.at[g], idx_v)
        pltpu.sync_copy(d2_hbm.at[g], d2_v)
        for p in range(_BC // _NB):
            _scan_pass(range(p * _NB, (p + 1) * _NB))
        pltpu.sync_copy(out_v, out_hbm.at[g])


@functools.cache
def _dtm_sc():
    return functools.partial(
        pl.kernel,
        out_type=jax.ShapeDtypeStruct((_NG, _BC, _L), jnp.float32),
        compiler_params=pltpu.CompilerParams(
            needs_layout_passes=False, use_tc_tiling_on_sc=False,
        ),
        mesh=plsc.VectorSubcoreMesh(
            core_axis_name="c", subcore_axis_name="s",
            num_cores=_NC, num_subcores=_NS,
        ),
        scratch_types=[
            pltpu.VMEM((_BC, _HW), jnp.float32),
            pltpu.VMEM((_HW, _L), jnp.int32),
            pltpu.VMEM((_HW, _L), jnp.float32),
            pltpu.VMEM((_BC, _L), jnp.float32),
            pltpu.VMEM((_BC, _L), jnp.float32),
        ],
    )(_dtm_body)


def kernel(input):
    b, c, h, w = input.shape
    weight = input.reshape(_BC, _HW)
    bounds = _bounds_tc(weight)
    dtm = _dtm_sc()(jnp.asarray(_IDX_T), jnp.asarray(_D2_T), weight, bounds)
    return dtm.transpose(1, 0, 2).reshape(b, c, h, w)
